# Initial kernel scaffold; baseline (speedup 1.0000x reference)
import jax
import jax.numpy as jnp
from jax.experimental import pallas as pl


def kernel(x, edge_index, W1, b1, W2, b2):
    raise NotImplementedError("write your pallas kernel here")


# speedup vs baseline: 23.0284x; 23.0284x over previous
import functools

import jax
import jax.numpy as jnp
from jax import lax
from jax.experimental import pallas as pl
from jax.experimental.pallas import tpu as pltpu
from jax.experimental.pallas import tpu_sc as plsc

N = 10000
E = 320000
D = 16
NT = 16
NP = 10240
NPT = NP // NT
CW = 125
ROWS = E // CW
RPT = ROWS // NT
IB = 8
NB = RPT // IB

_MESH = plsc.VectorSubcoreMesh(core_axis_name="c", subcore_axis_name="s",
                               num_cores=1)


def _zero_rows(ref, n):
  z = jnp.zeros((D,), jnp.float32)

  def body(i, c):
    ref[i, :] = z
    return c

  lax.fori_loop(0, n, body, 0)


@functools.partial(
    pl.kernel,
    out_type=jax.ShapeDtypeStruct((NP,), jnp.float32),
    mesh=_MESH,
    scratch_types=[
        pltpu.VMEM((IB, CW), jnp.int32),
        pltpu.VMEM((NP,), jnp.float32),
        pltpu.VMEM((NPT,), jnp.float32),
        pltpu.VMEM_SHARED((NT, NP), jnp.float32),
    ],
    compiler_params=pltpu.CompilerParams(needs_layout_passes=False, use_tc_tiling_on_sc=False),
)
def _deg_kernel(dst_hbm, deg_hbm, idx_v, deg_v, tmp_v, hist_sh):
  t = lax.axis_index("s")
  z = jnp.zeros((16,), jnp.float32)

  def zero(i, c):
    deg_v[pl.ds(i * 16, 16)] = z
    return c

  lax.fori_loop(0, NP // 16, zero, 0)

  ones = jnp.ones((16,), jnp.float32)
  ebase = t * RPT

  tail_mask = jnp.arange(16, dtype=jnp.int32) >= (16 - CW % 16)

  def blk(b, c):
    pltpu.sync_copy(dst_hbm.at[pl.ds(ebase + b * IB, IB)], idx_v)
    for j in range(IB):
      for k in range(CW // 16):
        idx = idx_v[j, pl.ds(k * 16, 16)]
        plsc.addupdate_scatter(deg_v, [idx], ones)
      idx = idx_v[j, pl.ds(CW - 16, 16)]
      plsc.addupdate_scatter(deg_v, [idx], ones, mask=tail_mask)
    return c

  lax.fori_loop(0, NB, blk, 0)
  pltpu.sync_copy(deg_v, hist_sh.at[t])
  plsc.subcore_barrier()

  nb = t * NPT
  for r in range(NT):
    pltpu.sync_copy(hist_sh.at[r, pl.ds(nb, NPT)], tmp_v)

    def acc(i, c):
      sl = pl.ds(i * 16, 16)
      if r == 0:
        deg_v[sl] = tmp_v[sl]
      else:
        deg_v[sl] = deg_v[sl] + tmp_v[sl]
      return c

    lax.fori_loop(0, NPT // 16, acc, 0)
  pltpu.sync_copy(deg_v.at[pl.ds(0, NPT)], deg_hbm.at[pl.ds(nb, NPT)])


@functools.partial(
    pl.kernel,
    out_type=(
        jax.ShapeDtypeStruct((NP, D), jnp.float32),
        jax.ShapeDtypeStruct((NP, D), jnp.float32),
        jax.ShapeDtypeStruct((NP, D), jnp.float32),
    ),
    mesh=_MESH,
    scratch_types=[
        pltpu.VMEM((NPT,), jnp.float32),
        pltpu.VMEM((NPT, D), jnp.float32),
        pltpu.VMEM((NPT, D), jnp.float32),
        pltpu.VMEM((IB, CW), jnp.int32),
        pltpu.VMEM((IB, CW), jnp.int32),
        pltpu.VMEM((IB, CW, D), jnp.float32),
        pltpu.VMEM((D,), jnp.float32),
        pltpu.VMEM_SHARED((NP, D), jnp.float32),
        pltpu.VMEM_SHARED((NP, D), jnp.float32),
    ],
    compiler_params=pltpu.CompilerParams(needs_layout_passes=False, use_tc_tiling_on_sc=False),
)
def _prop_kernel(h1_hbm, deg_hbm, src_hbm, dst_hbm, b1_hbm,
                 p2_hbm, g1_hbm, g2_hbm,
                 dinv_v, g_v, s_v, si_v, di_v, rows_v, b1_v, s1_sh, s2_sh):
  t = lax.axis_index("s")
  nb = t * NPT
  ebase = t * RPT

  pltpu.sync_copy(b1_hbm, b1_v)
  pltpu.sync_copy(deg_hbm.at[pl.ds(nb, NPT)], dinv_v)

  def newton(i, c):
    d = dinv_v[pl.ds(i * 16, 16)] + 1.0
    bits = plsc.bitcast(d, jnp.int32)
    bits = jnp.int32(0x5F3759DF) - lax.shift_right_logical(bits, 1)
    y = plsc.bitcast(bits, jnp.float32)
    y = y * (1.5 - 0.5 * d * y * y)
    y = y * (1.5 - 0.5 * d * y * y)
    y = y * (1.5 - 0.5 * d * y * y)
    dinv_v[pl.ds(i * 16, 16)] = y
    return c

  lax.fori_loop(0, NPT // 16, newton, 0)

  def _splat(dvec, j):
    return dvec.at[jnp.full((16,), j, jnp.int32)].get(
        mode="promise_in_bounds")

  pltpu.sync_copy(h1_hbm.at[pl.ds(nb, NPT)], g_v)

  def scale(k, c):
    dvec = dinv_v[pl.ds(k * 16, 16)]
    for j in range(16):
      row = k * 16 + j
      g_v[row, :] = g_v[row, :] * _splat(dvec, j)
    return c

  lax.fori_loop(0, NPT // 16, scale, 0)
  pltpu.sync_copy(g_v, g1_hbm.at[pl.ds(nb, NPT)])

  _zero_rows(s_v, NPT)
  pltpu.sync_copy(s_v, s1_sh.at[pl.ds(nb, NPT)])
  pltpu.sync_copy(s_v, s2_sh.at[pl.ds(nb, NPT)])
  plsc.subcore_barrier()

  def propagate(gtab_hbm, s_sh):
    def blk(b, c):
      pltpu.sync_copy(src_hbm.at[pl.ds(ebase + b * IB, IB)], si_v)
      pltpu.sync_copy(dst_hbm.at[pl.ds(ebase + b * IB, IB)], di_v)
      for j in range(IB):
        pltpu.sync_copy(gtab_hbm.at[si_v.at[j]], rows_v.at[j])
        pltpu.sync_copy(rows_v.at[j], s_sh.at[di_v.at[j]], add=True)
      return c

    lax.fori_loop(0, NB, blk, 0)

  propagate(g1_hbm, s1_sh)
  plsc.subcore_barrier()

  pltpu.sync_copy(s1_sh.at[pl.ds(nb, NPT)], s_v)
  b1v = b1_v[...]

  def mid(k, c):
    dvec = dinv_v[pl.ds(k * 16, 16)]
    for j in range(16):
      row = k * 16 + j
      dj = _splat(dvec, j)
      r = (s_v[row, :] + g_v[row, :]) * dj + b1v
      g_v[row, :] = jnp.maximum(r, 0.0) * dj
    return c

  lax.fori_loop(0, NPT // 16, mid, 0)
  pltpu.sync_copy(g_v, g2_hbm.at[pl.ds(nb, NPT)])
  plsc.subcore_barrier()

  propagate(g2_hbm, s2_sh)
  plsc.subcore_barrier()

  pltpu.sync_copy(s2_sh.at[pl.ds(nb, NPT)], s_v)

  def fin(k, c):
    dvec = dinv_v[pl.ds(k * 16, 16)]
    for j in range(16):
      row = k * 16 + j
      s_v[row, :] = (s_v[row, :] + g_v[row, :]) * _splat(dvec, j)
    return c

  lax.fori_loop(0, NPT // 16, fin, 0)
  pltpu.sync_copy(s_v, p2_hbm.at[pl.ds(nb, NPT)])


_BM = 1024


def _mm1_body(x_ref, w_ref, o_ref):
  o_ref[...] = jnp.dot(x_ref[...], w_ref[...],
                       preferred_element_type=jnp.float32)


def _mm2_body(p_ref, w_ref, b_ref, o_ref):
  o_ref[...] = jnp.dot(p_ref[...], w_ref[...],
                       preferred_element_type=jnp.float32) + b_ref[...]


def _mm1(xp, W1):
  return pl.pallas_call(
      _mm1_body,
      grid=(NP // _BM,),
      in_specs=[
          pl.BlockSpec((_BM, 128), lambda i: (i, 0)),
          pl.BlockSpec((128, D), lambda i: (0, 0)),
      ],
      out_specs=pl.BlockSpec((_BM, D), lambda i: (i, 0)),
      out_shape=jax.ShapeDtypeStruct((NP, D), jnp.float32),
  )(xp, W1)


def _mm2(p2, W2, b2):
  return pl.pallas_call(
      _mm2_body,
      grid=(NP // _BM,),
      in_specs=[
          pl.BlockSpec((_BM, D), lambda i: (i, 0)),
          pl.BlockSpec((D, 128), lambda i: (0, 0)),
          pl.BlockSpec((1, 128), lambda i: (0, 0)),
      ],
      out_specs=pl.BlockSpec((_BM, 128), lambda i: (i, 0)),
      out_shape=jax.ShapeDtypeStruct((NP, 128), jnp.float32),
  )(p2, W2, b2)


@jax.jit
def kernel(x, edge_index, W1, b1, W2, b2):
  xp = jnp.pad(x, ((0, NP - N), (0, 0)))
  src = edge_index[0].reshape(ROWS, CW)
  dst = edge_index[1].reshape(ROWS, CW)
  deg = _deg_kernel(dst)
  h1 = _mm1(xp, W1)
  p2, _, _ = _prop_kernel(h1, deg, src, dst, b1)
  out = _mm2(p2, W2, b2[None, :])
  return out[:N]


# speedup vs baseline: 48.6707x; 2.1135x over previous
import functools

import jax
import jax.numpy as jnp
from jax import lax
from jax.experimental import pallas as pl
from jax.experimental.pallas import tpu as pltpu
from jax.experimental.pallas import tpu_sc as plsc

N = 10000
E = 320000
D = 16
NT = 16
NP = 10240
NPT = NP // NT
CW = 125
ROWS = E // CW
RPT = ROWS // NT
IB = 8
NB = RPT // IB
GS = 4
NG = RPT // GS

_MESH = plsc.VectorSubcoreMesh(core_axis_name="c", subcore_axis_name="s",
                               num_cores=1)


def _zero_rows(ref, n):
  z = jnp.zeros((D,), jnp.float32)

  def body(i, c):
    ref[i, :] = z
    return c

  lax.fori_loop(0, n, body, 0)


@functools.partial(
    pl.kernel,
    out_type=jax.ShapeDtypeStruct((NP,), jnp.float32),
    mesh=_MESH,
    scratch_types=[
        pltpu.VMEM((IB, CW), jnp.int32),
        pltpu.VMEM((NP,), jnp.float32),
        pltpu.VMEM((NPT,), jnp.float32),
        pltpu.VMEM_SHARED((NT, NP), jnp.float32),
    ],
    compiler_params=pltpu.CompilerParams(needs_layout_passes=False, use_tc_tiling_on_sc=False),
)
def _deg_kernel(dst_hbm, deg_hbm, idx_v, deg_v, tmp_v, hist_sh):
  t = lax.axis_index("s")
  z = jnp.zeros((16,), jnp.float32)

  def zero(i, c):
    deg_v[pl.ds(i * 16, 16)] = z
    return c

  lax.fori_loop(0, NP // 16, zero, 0)

  ones = jnp.ones((16,), jnp.float32)
  ebase = t * RPT

  tail_mask = jnp.arange(16, dtype=jnp.int32) >= (16 - CW % 16)

  def blk(b, c):
    pltpu.sync_copy(dst_hbm.at[pl.ds(ebase + b * IB, IB)], idx_v)
    for j in range(IB):
      for k in range(CW // 16):
        idx = idx_v[j, pl.ds(k * 16, 16)]
        plsc.addupdate_scatter(deg_v, [idx], ones)
      idx = idx_v[j, pl.ds(CW - 16, 16)]
      plsc.addupdate_scatter(deg_v, [idx], ones, mask=tail_mask)
    return c

  lax.fori_loop(0, NB, blk, 0)
  pltpu.sync_copy(deg_v, hist_sh.at[t])
  plsc.subcore_barrier()

  nb = t * NPT
  for r in range(NT):
    pltpu.sync_copy(hist_sh.at[r, pl.ds(nb, NPT)], tmp_v)

    def acc(i, c):
      sl = pl.ds(i * 16, 16)
      if r == 0:
        deg_v[sl] = tmp_v[sl]
      else:
        deg_v[sl] = deg_v[sl] + tmp_v[sl]
      return c

    lax.fori_loop(0, NPT // 16, acc, 0)
  pltpu.sync_copy(deg_v.at[pl.ds(0, NPT)], deg_hbm.at[pl.ds(nb, NPT)])


@functools.partial(
    pl.kernel,
    out_type=(
        jax.ShapeDtypeStruct((NP, D), jnp.float32),
        jax.ShapeDtypeStruct((NP, D), jnp.float32),
        jax.ShapeDtypeStruct((NP, D), jnp.float32),
    ),
    mesh=_MESH,
    scratch_types=[
        pltpu.VMEM((NPT,), jnp.float32),
        pltpu.VMEM((NPT, D), jnp.float32),
        pltpu.VMEM((NPT, D), jnp.float32),
        pltpu.VMEM((RPT, CW), jnp.int32),
        pltpu.VMEM((RPT, CW), jnp.int32),
        pltpu.VMEM((2, GS, CW, D), jnp.float32),
        pltpu.VMEM((D,), jnp.float32),
        pltpu.SemaphoreType.DMA,
        pltpu.SemaphoreType.DMA,
        pltpu.SemaphoreType.DMA,
        pltpu.VMEM_SHARED((NP, D), jnp.float32),
        pltpu.VMEM_SHARED((NP, D), jnp.float32),
    ],
    compiler_params=pltpu.CompilerParams(needs_layout_passes=False, use_tc_tiling_on_sc=False),
)
def _prop_kernel(h1_hbm, deg_hbm, src_hbm, dst_hbm, b1_hbm,
                 p2_hbm, g1_hbm, g2_hbm,
                 dinv_v, g_v, s_v, si_v, di_v, rows_v, b1_v,
                 sem_g, sem_s0, sem_s1, s1_sh, s2_sh):
  t = lax.axis_index("s")
  nb = t * NPT
  ebase = t * RPT

  pltpu.sync_copy(b1_hbm, b1_v)
  pltpu.sync_copy(deg_hbm.at[pl.ds(nb, NPT)], dinv_v)
  pltpu.sync_copy(src_hbm.at[pl.ds(ebase, RPT)], si_v)
  pltpu.sync_copy(dst_hbm.at[pl.ds(ebase, RPT)], di_v)

  def newton(i, c):
    d = dinv_v[pl.ds(i * 16, 16)] + 1.0
    bits = plsc.bitcast(d, jnp.int32)
    bits = jnp.int32(0x5F3759DF) - lax.shift_right_logical(bits, 1)
    y = plsc.bitcast(bits, jnp.float32)
    y = y * (1.5 - 0.5 * d * y * y)
    y = y * (1.5 - 0.5 * d * y * y)
    y = y * (1.5 - 0.5 * d * y * y)
    dinv_v[pl.ds(i * 16, 16)] = y
    return c

  lax.fori_loop(0, NPT // 16, newton, 0)

  def _splat(dvec, j):
    return dvec.at[jnp.full((16,), j, jnp.int32)].get(
        mode="promise_in_bounds")

  pltpu.sync_copy(h1_hbm.at[pl.ds(nb, NPT)], g_v)

  def scale(k, c):
    dvec = dinv_v[pl.ds(k * 16, 16)]
    for j in range(16):
      row = k * 16 + j
      g_v[row, :] = g_v[row, :] * _splat(dvec, j)
    return c

  lax.fori_loop(0, NPT // 16, scale, 0)
  pltpu.sync_copy(g_v, g1_hbm.at[pl.ds(nb, NPT)])

  _zero_rows(s_v, NPT)
  pltpu.sync_copy(s_v, s1_sh.at[pl.ds(nb, NPT)])
  pltpu.sync_copy(s_v, s2_sh.at[pl.ds(nb, NPT)])
  plsc.subcore_barrier()

  def propagate(gtab_hbm, s_sh):
    sems = [sem_s0, sem_s1]

    def pair(p, c):
      for par in range(2):
        grp = p * 2 + par
        buf = rows_v.at[par]
        sem_s = sems[par]

        @pl.when(grp >= 2)
        def _drain():
          for j in range(GS):
            pltpu.make_async_copy(buf.at[j], s_sh.at[di_v.at[0]],
                                  sem_s).wait()

        gs = []
        for j in range(GS):
          row = grp * GS + j
          gs.append(pltpu.async_copy(gtab_hbm.at[si_v.at[row]], buf.at[j],
                                     sem_g))
        for gcopy in gs:
          gcopy.wait()
        for j in range(GS):
          row = grp * GS + j
          pltpu.async_copy(buf.at[j], s_sh.at[di_v.at[row]], sem_s, add=True)
      return c

    lax.fori_loop(0, NG // 2, pair, 0)
    for par in range(2):
      for j in range(GS):
        pltpu.make_async_copy(rows_v.at[par].at[j], s_sh.at[di_v.at[0]],
                              sems[par]).wait()

  propagate(g1_hbm, s1_sh)
  plsc.subcore_barrier()

  pltpu.sync_copy(s1_sh.at[pl.ds(nb, NPT)], s_v)
  b1v = b1_v[...]

  def mid(k, c):
    dvec = dinv_v[pl.ds(k * 16, 16)]
    for j in range(16):
      row = k * 16 + j
      dj = _splat(dvec, j)
      r = (s_v[row, :] + g_v[row, :]) * dj + b1v
      g_v[row, :] = jnp.maximum(r, 0.0) * dj
    return c

  lax.fori_loop(0, NPT // 16, mid, 0)
  pltpu.sync_copy(g_v, g2_hbm.at[pl.ds(nb, NPT)])
  plsc.subcore_barrier()

  propagate(g2_hbm, s2_sh)
  plsc.subcore_barrier()

  pltpu.sync_copy(s2_sh.at[pl.ds(nb, NPT)], s_v)

  def fin(k, c):
    dvec = dinv_v[pl.ds(k * 16, 16)]
    for j in range(16):
      row = k * 16 + j
      s_v[row, :] = (s_v[row, :] + g_v[row, :]) * _splat(dvec, j)
    return c

  lax.fori_loop(0, NPT // 16, fin, 0)
  pltpu.sync_copy(s_v, p2_hbm.at[pl.ds(nb, NPT)])


_BM = 1024


def _mm1_body(x_ref, w_ref, o_ref):
  o_ref[...] = jnp.dot(x_ref[...], w_ref[...],
                       preferred_element_type=jnp.float32)


def _mm2_body(p_ref, w_ref, b_ref, o_ref):
  o_ref[...] = jnp.dot(p_ref[...], w_ref[...],
                       preferred_element_type=jnp.float32) + b_ref[...]


def _mm1(xp, W1):
  return pl.pallas_call(
      _mm1_body,
      grid=(NP // _BM,),
      in_specs=[
          pl.BlockSpec((_BM, 128), lambda i: (i, 0)),
          pl.BlockSpec((128, D), lambda i: (0, 0)),
      ],
      out_specs=pl.BlockSpec((_BM, D), lambda i: (i, 0)),
      out_shape=jax.ShapeDtypeStruct((NP, D), jnp.float32),
  )(xp, W1)


def _mm2(p2, W2, b2):
  return pl.pallas_call(
      _mm2_body,
      grid=(NP // _BM,),
      in_specs=[
          pl.BlockSpec((_BM, D), lambda i: (i, 0)),
          pl.BlockSpec((D, 128), lambda i: (0, 0)),
          pl.BlockSpec((1, 128), lambda i: (0, 0)),
      ],
      out_specs=pl.BlockSpec((_BM, 128), lambda i: (i, 0)),
      out_shape=jax.ShapeDtypeStruct((NP, 128), jnp.float32),
  )(p2, W2, b2)


@jax.jit
def kernel(x, edge_index, W1, b1, W2, b2):
  xp = jnp.pad(x, ((0, NP - N), (0, 0)))
  src = edge_index[0].reshape(ROWS, CW)
  dst = edge_index[1].reshape(ROWS, CW)
  deg = _deg_kernel(dst)
  h1 = _mm1(xp, W1)
  p2, _, _ = _prop_kernel(h1, deg, src, dst, b1)
  out = _mm2(p2, W2, b2[None, :])
  return out[:N]
